# Initial kernel scaffold; baseline (speedup 1.0000x reference)
import jax
import jax.numpy as jnp
from jax.experimental import pallas as pl


def kernel(observation, goal, action, conv1_w, conv1_b, conv2_w, conv2_b, conv3_w, conv3_b, conv4_w, conv4_b, conv5_w, conv5_b, conv6_w, conv6_b, conv7_w, conv7_b, w_obs, b_obs, w_goal, b_goal, w1a, w1b, b1, w_act, b_act, w2a, w2b, b2, w3, b3):
    raise NotImplementedError("write your pallas kernel here")


# speedup vs baseline: 55.2301x; 55.2301x over previous
import jax
import jax.numpy as jnp
from jax.experimental import pallas as pl
from jax.experimental.pallas import tpu as pltpu

NEG_SLOPE = 0.01
G = 16


def _lrelu(v):
    return jnp.maximum(v, NEG_SLOPE * v)


def _shift_y(t, s):
    z = jnp.zeros_like(t[:, :1, :])
    if s == 1:
        return jnp.concatenate([t[:, 1:, :], z], axis=1)
    return jnp.concatenate([z, t[:, :-1, :]], axis=1)


def _shift_x(t, s):
    z = jnp.zeros_like(t[:, :, :G])
    if s == 1:
        return jnp.concatenate([t[:, :, G:], z], axis=2)
    return jnp.concatenate([z, t[:, :, :-G]], axis=2)


def _conv_s2(A, W, b, k, pad):
    Ci, Fi, _, Y, L = A.shape
    Fo = Fi // 2
    Co = W.shape[0]
    outs = []
    for fy in range(Fo):
        for fx in range(Fo):
            slabs = []
            for i in range(k):
                for j in range(k):
                    fin_y, sy = 2 * fy + i - pad, 0
                    if fin_y < 0:
                        fin_y, sy = fin_y + Fi, -1
                    elif fin_y >= Fi:
                        fin_y, sy = fin_y - Fi, 1
                    fin_x, sx = 2 * fx + j - pad, 0
                    if fin_x < 0:
                        fin_x, sx = fin_x + Fi, -1
                    elif fin_x >= Fi:
                        fin_x, sx = fin_x - Fi, 1
                    t = A[:, fin_y, fin_x]
                    if sy:
                        t = _shift_y(t, sy)
                    if sx:
                        t = _shift_x(t, sx)
                    slabs.append(t)
            p = jnp.concatenate(slabs, axis=0)
            y = jnp.einsum('nk,kyl->nyl', W, p)
            outs.append(_lrelu(y + b))
    out = jnp.stack(outs, axis=1)
    return out.reshape(Co, Fo, Fo, Y, L)


def _conv_s1_3x3(A, W, b):
    slabs = []
    for i in range(3):
        for j in range(3):
            t = A
            if i != 1:
                t = _shift_y(t, i - 1)
            if j != 1:
                t = _shift_x(t, j - 1)
            slabs.append(t)
    p = jnp.concatenate(slabs, axis=0)
    return _lrelu(jnp.einsum('nk,kyl->nyl', W, p) + b)


def _conv_stack_kernel(x_ref, w1, b1, w2, b2, w3, b3, w4, b4, w5, b5,
                       w6, b6, w7, b7, o_ref):
    A = x_ref[0]
    A = _conv_s2(A, w1[...], b1[...], 5, 2)
    A = _conv_s2(A, w2[...], b2[...], 3, 1)
    A = _conv_s2(A, w3[...], b3[...], 3, 1)
    A = _conv_s2(A, w4[...], b4[...], 3, 1)
    A = _conv_s2(A, w5[...], b5[...], 3, 1)
    A = A[:, 0, 0]
    A = _lrelu(jnp.einsum('nc,cyl->nyl', w6[...], A) + b6[...])
    A = _conv_s1_3x3(A, w7[...], b7[...])
    o_ref[0] = A


def _mlp_tail_kernel(obs_ref, goal_ref, act_ref,
                     wo_ref, bo_ref, wg_ref, bg_ref,
                     w1a_ref, w1b_ref, b1_ref, wa_ref, ba_ref,
                     w2a_ref, w2b_ref, b2_ref, w3_ref, b3_ref, out_ref):
    def dot(a, bm):
        return jnp.dot(a, bm, preferred_element_type=jnp.float32)

    h_obs = _lrelu(dot(obs_ref[...], wo_ref[...]) + bo_ref[...])
    h_goal = _lrelu(dot(goal_ref[...], wg_ref[...]) + bg_ref[...])
    state = _lrelu(dot(h_obs, w1a_ref[...]) + dot(h_goal, w1b_ref[...])
                   + b1_ref[...])
    h_act = _lrelu(dot(act_ref[...], wa_ref[...]) + ba_ref[...])
    h = _lrelu(dot(state, w2a_ref[...]) + dot(h_act, w2b_ref[...])
               + b2_ref[...])
    out_ref[...] = _lrelu(dot(h, w3_ref[...]) + b3_ref[...])


def _perm_w(w_flat, ci, k):
    co = w_flat.shape[0]
    return (w_flat.reshape(co, ci, k, k).transpose(0, 2, 3, 1)
            .reshape(co, k * k * ci))


def kernel(observation, goal, action, conv1_w, conv1_b, conv2_w, conv2_b,
           conv3_w, conv3_b, conv4_w, conv4_b, conv5_w, conv5_b, conv6_w,
           conv6_b, conv7_w, conv7_b, w_obs, b_obs, w_goal, b_goal, w1a, w1b,
           b1, w_act, b_act, w2a, w2b, b2, w3, b3):
    B = observation.shape[0]
    ng = B // G
    X = (observation.reshape(ng, G, 3, 8, 32, 8, 32)
         .transpose(0, 2, 4, 6, 3, 5, 1)
         .reshape(ng, 3, 32, 32, 8, 8 * G))

    ws = [
        (_perm_w(conv1_w, 3, 5), conv1_b.reshape(8, 1, 1)),
        (_perm_w(conv2_w, 8, 3), conv2_b.reshape(16, 1, 1)),
        (_perm_w(conv3_w, 16, 3), conv3_b.reshape(32, 1, 1)),
        (_perm_w(conv4_w, 32, 3), conv4_b.reshape(64, 1, 1)),
        (_perm_w(conv5_w, 64, 3), conv5_b.reshape(128, 1, 1)),
        (conv6_w, conv6_b.reshape(32, 1, 1)),
        (_perm_w(conv7_w, 32, 3), conv7_b.reshape(32, 1, 1)),
    ]
    flat_ws = [a for pair in ws for a in pair]

    def const_spec(arr):
        nd = arr.ndim
        return pl.BlockSpec(arr.shape, lambda g, _n=nd: (0,) * _n)

    conv_out = pl.pallas_call(
        _conv_stack_kernel,
        out_shape=jax.ShapeDtypeStruct((ng, 32, 8, 8 * G), jnp.float32),
        grid_spec=pltpu.PrefetchScalarGridSpec(
            num_scalar_prefetch=0,
            grid=(ng,),
            in_specs=[pl.BlockSpec((1, 3, 32, 32, 8, 8 * G),
                                   lambda g: (g, 0, 0, 0, 0, 0))]
                     + [const_spec(a) for a in flat_ws],
            out_specs=pl.BlockSpec((1, 32, 8, 8 * G),
                                   lambda g: (g, 0, 0, 0)),
        ),
        compiler_params=pltpu.CompilerParams(
            dimension_semantics=("parallel",),
            vmem_limit_bytes=56 * 1024 * 1024),
    )(X, *flat_ws)

    obs_flat = (conv_out.reshape(ng, 32, 8, 8, G)
                .transpose(0, 4, 1, 2, 3)
                .reshape(B, 2048))

    out = pl.pallas_call(
        _mlp_tail_kernel,
        out_shape=jax.ShapeDtypeStruct((B, 1), jnp.float32),
    )(obs_flat, goal, action, w_obs, b_obs, w_goal, b_goal,
      w1a, w1b, b1, w_act, b_act, w2a, w2b, b2, w3, b3)
    return out
